# baseline (speedup 1.0000x reference)
import functools
import math

import jax
import jax.numpy as jnp
from jax import lax
from jax.experimental import pallas as pl
from jax.experimental.pallas import tpu as pltpu
from jax.experimental.pallas import tpu_sc as plsc

_VOCAB = 100000
_SMOOTHING = 0.1
_CONF = 1.0 - _SMOOTHING
_EPS = _SMOOTHING / (_VOCAB - 2)
_C1 = (_VOCAB - 2) * _EPS * math.log(_EPS) + _CONF * math.log(_CONF)

_N = 1024
_NW = 32
_NSC = 256
_ROW0 = _N - _NSC
_RPW = _NSC // _NW
_CW = 20000
_CPR = _VOCAB // _CW
_NSLOT = 5
_UN = 25
_NACC = 5


@functools.partial(
    pl.kernel,
    out_type=jax.ShapeDtypeStruct((_NW * 16,), jnp.float32),
    mesh=plsc.VectorSubcoreMesh(core_axis_name="c", subcore_axis_name="s"),
    scratch_types=[
        pltpu.VMEM((_CW,), jnp.float32),
        pltpu.VMEM((_CW,), jnp.float32),
        pltpu.VMEM((_CW,), jnp.float32),
        pltpu.VMEM((_CW,), jnp.float32),
        pltpu.VMEM((_CW,), jnp.float32),
        pltpu.VMEM((128,), jnp.int32),
        pltpu.VMEM((16,), jnp.float32),
        pltpu.SemaphoreType.DMA,
        pltpu.SemaphoreType.DMA,
        pltpu.SemaphoreType.DMA,
        pltpu.SemaphoreType.DMA,
        pltpu.SemaphoreType.DMA,
    ],
)
def _sc_loss(
    x_hbm, t_hbm, out_hbm, b0, b1, b2, b3, b4, tvals, outbuf, s0, s1, s2, s3, s4
):
    wid = lax.axis_index("s") * 2 + lax.axis_index("c")
    row0 = _ROW0 + wid * _RPW

    pltpu.sync_copy(t_hbm.at[pl.ds(row0, _RPW)], tvals.at[pl.ds(0, _RPW)])

    bufs = (b0, b1, b2, b3, b4)
    sems = (s0, s1, s2, s3, s4)

    def copy(seg, slot):
        r = seg // _CPR
        c = (seg % _CPR) * _CW
        return pltpu.make_async_copy(
            x_hbm.at[pl.ds((row0 + r) * _VOCAB + c, _CW)],
            bufs[slot],
            sems[slot],
        )

    for pre in range(_NSLOT - 1):
        copy(pre, pre).start()

    lanes = lax.iota(jnp.int32, 16)
    onehot0 = jnp.where(lanes == 0, 1.0, 0.0).astype(jnp.float32)
    nseg = _RPW * _CPR

    def row_body(r, total_vec):
        t_r = tvals[pl.ds(r, 16)][0]

        acc = tuple(jnp.zeros((16,), jnp.float32) for _ in range(_NACC))
        gacc = jnp.zeros((16,), jnp.float32)
        x0vec = jnp.zeros((16,), jnp.float32)
        for cc in range(_CPR):
            seg = r * _CPR + cc
            nxt = seg + _NSLOT - 1

            @pl.when(nxt < nseg)
            def _start():
                copy(nxt, (cc + _NSLOT - 1) % _NSLOT).start()

            copy(seg, cc).wait()
            buf = bufs[cc]

            if cc == 0:
                x0vec = onehot0 * buf[pl.ds(0, 16)]

            def body(k, a):
                base = k * (16 * _UN)
                a = list(a)
                for u in range(_UN):
                    a[u % _NACC] = a[u % _NACC] + buf[pl.ds(base + u * 16, 16)]
                return tuple(a)

            acc = lax.fori_loop(0, _CW // (16 * _UN), body, acc)

            lo = cc * _CW
            inb = (t_r >= lo) & (t_r < lo + _CW)
            iv = jnp.clip(t_r - lo, 0, _CW - 1)
            iv16 = (iv // 16) * 16
            gvec = buf[pl.ds(iv16, 16)]
            ghit = jnp.where(lanes == (iv - iv16), gvec, 0.0)
            gacc = jnp.where(inb, ghit, gacc)

        s_vec = sum(acc[1:], acc[0])
        m_f = jnp.where(t_r != 0, jnp.float32(1.0), jnp.float32(0.0))
        vrow = (
            _C1 * onehot0
            + _EPS * x0vec
            - (_CONF - _EPS) * gacc
            - _EPS * s_vec
        )
        return total_vec + m_f * vrow

    total_vec = lax.fori_loop(0, _RPW, row_body, jnp.zeros((16,), jnp.float32))

    outbuf[...] = total_vec
    pltpu.sync_copy(outbuf, out_hbm.at[pl.ds(wid * 16, 16)])


_BC = 4096
_NBLK = (_VOCAB + _BC - 1) // _BC
_NTC = _ROW0


def _kl_body(x_ref, t_ref, o_ref, sacc, gacc, x0):
    j = pl.program_id(0)

    @pl.when(j == 0)
    def _init():
        sacc[...] = jnp.zeros_like(sacc)
        gacc[...] = jnp.zeros_like(gacc)
        x0[...] = x_ref[:, 0:1]

    xb = x_ref[...]
    idx = t_ref[...] - j * _BC
    lane = jax.lax.broadcasted_iota(jnp.int32, (_NTC, _BC), 1)
    hit = lane == idx
    gacc[...] += jnp.sum(jnp.where(hit, xb, 0.0), axis=1, keepdims=True)

    @pl.when(j < _NBLK - 1)
    def _full():
        sacc[...] += jnp.sum(xb, axis=1, keepdims=True)

    @pl.when(j == _NBLK - 1)
    def _tail():
        valid = lane < (_VOCAB - (_NBLK - 1) * _BC)
        sacc[...] += jnp.sum(jnp.where(valid, xb, 0.0), axis=1, keepdims=True)

        m = t_ref[...] != 0
        wsum = _EPS * (sacc[...] - x0[...]) + (_CONF - _EPS) * gacc[...]
        o_ref[0, 0] = jnp.sum(jnp.where(m, _C1 - wsum, 0.0))


def _tc_loss(x, tgt2):
    return pl.pallas_call(
        _kl_body,
        grid=(_NBLK,),
        in_specs=[
            pl.BlockSpec((_NTC, _BC), lambda j: (0, j)),
            pl.BlockSpec((_NTC, 1), lambda j: (0, 0)),
        ],
        out_specs=pl.BlockSpec(memory_space=pltpu.SMEM),
        out_shape=jax.ShapeDtypeStruct((1, 1), jnp.float32),
        scratch_shapes=[
            pltpu.VMEM((_NTC, 1), jnp.float32),
            pltpu.VMEM((_NTC, 1), jnp.float32),
            pltpu.VMEM((_NTC, 1), jnp.float32),
        ],
    )(x, tgt2)


def _sum_body(p_ref, a_ref, o_ref):
    o_ref[0, 0] = a_ref[0, 0] + jnp.sum(p_ref[...])


def kernel(x, target):
    tgt = target.astype(jnp.int32)
    partials = _sc_loss(x.reshape(-1), tgt)
    a = _tc_loss(x, tgt.reshape(_N, 1)[:_NTC])
    out = pl.pallas_call(
        _sum_body,
        in_specs=[
            pl.BlockSpec((8, 64), lambda: (0, 0)),
            pl.BlockSpec(memory_space=pltpu.SMEM),
        ],
        out_specs=pl.BlockSpec(memory_space=pltpu.SMEM),
        out_shape=jax.ShapeDtypeStruct((1, 1), jnp.float32),
    )(partials.reshape(8, 64), a)
    return out[0, 0]


# speedup vs baseline: 2.2467x; 2.2467x over previous
import math

import jax
import jax.numpy as jnp
from jax.experimental import pallas as pl
from jax.experimental.pallas import tpu as pltpu

_VOCAB = 100000
_SMOOTHING = 0.1
_CONF = 1.0 - _SMOOTHING
_EPS = _SMOOTHING / (_VOCAB - 2)
_C1 = (_VOCAB - 2) * _EPS * math.log(_EPS) + _CONF * math.log(_CONF)

_N = 1024
_BC = 4096
_NBLK = (_VOCAB + _BC - 1) // _BC


def _kl_body(x_ref, t_ref, o_ref, sacc, gacc, x0):
    j = pl.program_id(0)

    @pl.when(j == 0)
    def _init():
        sacc[...] = jnp.zeros_like(sacc)
        gacc[...] = jnp.zeros_like(gacc)
        x0[...] = x_ref[:, 0:1]

    xb = x_ref[...]
    idx = t_ref[...] - j * _BC
    lane = jax.lax.broadcasted_iota(jnp.int32, (_N, _BC), 1)
    hit = lane == idx
    gacc[...] += jnp.sum(jnp.where(hit, xb, 0.0), axis=1, keepdims=True)

    @pl.when(j < _NBLK - 1)
    def _full():
        sacc[...] += jnp.sum(xb, axis=1, keepdims=True)

    @pl.when(j == _NBLK - 1)
    def _tail():
        valid = lane < (_VOCAB - (_NBLK - 1) * _BC)
        sacc[...] += jnp.sum(jnp.where(valid, xb, 0.0), axis=1, keepdims=True)

        m = t_ref[...] != 0
        wsum = _EPS * (sacc[...] - x0[...]) + (_CONF - _EPS) * gacc[...]
        o_ref[0, 0] = jnp.sum(jnp.where(m, _C1 - wsum, 0.0))


def kernel(x, target):
    n, v = x.shape
    tgt2 = target.astype(jnp.int32).reshape(n, 1)
    out = pl.pallas_call(
        _kl_body,
        grid=(_NBLK,),
        in_specs=[
            pl.BlockSpec((n, _BC), lambda j: (0, j)),
            pl.BlockSpec((n, 1), lambda j: (0, 0)),
        ],
        out_specs=pl.BlockSpec(memory_space=pltpu.SMEM),
        out_shape=jax.ShapeDtypeStruct((1, 1), jnp.float32),
        scratch_shapes=[
            pltpu.VMEM((n, 1), jnp.float32),
            pltpu.VMEM((n, 1), jnp.float32),
            pltpu.VMEM((n, 1), jnp.float32),
        ],
    )(x, tgt2)
    return out[0, 0]
